# baseline (speedup 1.0000x reference)
import functools

import jax
import jax.numpy as jnp
from jax import lax
from jax.experimental import pallas as pl
from jax.experimental.pallas import tpu as pltpu
from jax.experimental.pallas import tpu_sc as plsc

_B = 16384
_D = 32
_V = 1000000
_L = 16

_NC = 2
_NS = 16
_NW = _NC * _NS
_BPW = _B // _NW
_NBLK = _BPW // _L

_TW = 32768
_TGRID = (_V + _TW - 1) // _TW


def _tc_transpose_body(tT_ref, out_ref):
    out_ref[...] = tT_ref[...].T


def _transpose_table(tableT):
    return pl.pallas_call(
        _tc_transpose_body,
        grid=(_TGRID,),
        in_specs=[pl.BlockSpec((_D, _TW), lambda i: (0, i))],
        out_specs=pl.BlockSpec((_TW, _D), lambda i: (i, 0)),
        out_shape=jax.ShapeDtypeStruct((_V, _D), jnp.float32),
        compiler_params=pltpu.CompilerParams(
            dimension_semantics=("arbitrary",),
        ),
    )(tableT)


def _rsqrt_f32(x):
    i = plsc.bitcast(x, jnp.int32)
    i = jnp.int32(0x5F3759DF) - lax.shift_right_logical(i, 1)
    y = plsc.bitcast(i, jnp.float32)
    for _ in range(3):
        y = y * (1.5 - 0.5 * x * y * y)
    return y


def _sc_body(table_hbm, idx_hbm, outT_hbm, idx_v, rows_v, cols_v, sem):
    wid = lax.axis_index("s") * _NC + lax.axis_index("c")
    base = wid * _BPW
    pltpu.sync_copy(idx_hbm.at[pl.ds(base, _BPW)], idx_v)

    lanes = lax.iota(jnp.int32, _L)

    def gather_blk(i, carry):
        ch = idx_v[pl.ds(i * _L, _L)]
        for k in range(_L):
            pltpu.async_copy(
                table_hbm.at[pl.ds(ch[k], 1)],
                rows_v.at[pl.ds(i * _L + k, 1)],
                sem,
            )
        return carry

    lax.fori_loop(0, _NBLK, gather_blk, 0)
    pltpu.make_async_copy(table_hbm.at[pl.ds(0, _BPW)], rows_v, sem).wait()

    def block(i, carry):
        row_idx = i * _L + lanes
        acc = jnp.zeros((_L,), jnp.float32)
        vals = []
        for d in range(_D):
            col = jnp.full((_L,), d, jnp.int32)
            v = plsc.load_gather(rows_v, [row_idx, col])
            vals.append(v)
            acc = acc + v * v
        rinv = _rsqrt_f32(jnp.maximum(acc, jnp.float32(1e-24)))
        for d in range(_D):
            plsc.store_scatter(
                cols_v, [jnp.full((_L,), d, jnp.int32), row_idx], vals[d] * rinv
            )
        return carry

    lax.fori_loop(0, _NBLK, block, 0)
    pltpu.sync_copy(cols_v, outT_hbm.at[pl.ds(0, _D), pl.ds(base, _BPW)])


@jax.jit
def kernel(X, table):
    mesh = plsc.VectorSubcoreMesh(core_axis_name="c", subcore_axis_name="s")
    run = functools.partial(
        pl.kernel,
        mesh=mesh,
        compiler_params=pltpu.CompilerParams(needs_layout_passes=False),
        out_type=jax.ShapeDtypeStruct((_D, _B), jnp.float32),
        scratch_types=[
            pltpu.VMEM((_BPW,), jnp.int32),
            pltpu.VMEM((_BPW, _D), jnp.float32),
            pltpu.VMEM((_D, _BPW), jnp.float32),
            pltpu.SemaphoreType.DMA,
        ],
    )(_sc_body)
    table_rm = _transpose_table(table.T)
    outT = run(table_rm, X.astype(jnp.int32))
    return outT.T
